# baseline (speedup 1.0000x reference)
import functools

import jax
import jax.numpy as jnp
from jax import lax
from jax.experimental import pallas as pl
from jax.experimental.pallas import tpu as pltpu
from jax.experimental.pallas import tpu_sc as plsc

_B, _L, _E, _R, _D = 10, 1000, 320000, 512, 128
_N_ENT = _B * _L

_N_CORES = 2
_N_TILES = 16
_N_RANGES = 4
_ROWS = _N_ENT // _N_RANGES
_ACC_WORDS = _ROWS * _R
_ACC_PAD = _ACC_WORDS + 2048
_ZSLICE = _ACC_PAD // _N_TILES
_WSLICE = _ACC_WORDS // _N_TILES
_DUMMY = _ACC_WORDS

_CHUNK = 4096
_SUBROWS = _CHUNK // 128
_Q = 20480
_NCHUNK = _Q // _CHUNK
_E_PAD = _N_TILES * _Q


_sc_mesh = plsc.VectorSubcoreMesh(core_axis_name="c", subcore_axis_name="s")


@functools.partial(
    pl.kernel,
    out_type=jax.ShapeDtypeStruct((_N_RANGES, _ACC_WORDS), jnp.float32),
    mesh=_sc_mesh,
    scratch_types=[
        [pltpu.VMEM((_CHUNK,), jnp.int32)] * 2,
        [pltpu.VMEM((_CHUNK,), jnp.int32)] * 2,
        [pltpu.VMEM((_CHUNK,), jnp.int32)] * 2,
        [pltpu.VMEM((_SUBROWS, 128), jnp.float32)] * 2,
        [pltpu.VMEM((_SUBROWS, 128), jnp.int32)] * 2,
        [pltpu.VMEM((_SUBROWS, 128), jnp.int32)] * 2,
        pltpu.VMEM_SHARED((_ACC_PAD,), jnp.float32),
        [pltpu.SemaphoreType.DMA] * 2,
        [pltpu.SemaphoreType.DMA] * 2,
    ],
)
def _build_c(tails, heads, rels, w2d, zeros_hbm, out,
             tb, hb, rb, wb, ftb, fhb, acc, insem, scsem):
    c = lax.axis_index("c")
    s = lax.axis_index("s")

    def fire_inputs(k, p):
        off = s * _Q + k * _CHUNK
        row_off = pl.multiple_of(off // 128, 8)
        return [
            pltpu.async_copy(tails.at[pl.ds(off, _CHUNK)], tb[p], insem[p]),
            pltpu.async_copy(heads.at[pl.ds(off, _CHUNK)], hb[p], insem[p]),
            pltpu.async_copy(rels.at[pl.ds(off, _CHUNK)], rb[p], insem[p]),
            pltpu.async_copy(w2d.at[pl.ds(row_off, _SUBROWS)], wb[p], insem[p]),
        ]

    for rng in range(_N_RANGES // _N_CORES):
        rid = c * (_N_RANGES // _N_CORES) + rng
        base_row = rid * _ROWS

        pltpu.sync_copy(zeros_hbm,
                        acc.at[pl.ds(pl.multiple_of(s * _ZSLICE, 128), _ZSLICE)])
        plsc.subcore_barrier()

        for k in range(_NCHUNK):
            p = 0
            for d in fire_inputs(k, p):
                d.wait()

            t_buf, h_buf, r_buf = tb[p], hb[p], rb[p]
            ft_buf, fh_buf = ftb[p], fhb[p]

            def vec_body(i, _):
                j = i // 8
                l = i - j * 8
                sl = pl.ds(i * 16, 16)
                dsl = pl.ds(l * 16, 16)
                tv = t_buf[sl]
                hv = h_buf[sl]
                rv = r_buf[sl]
                lt = tv - base_row
                ft = jnp.where((lt >= 0) & (lt < _ROWS), lt * _R + rv, _DUMMY)
                lh = hv - base_row
                fh = jnp.where((lh >= 0) & (lh < _ROWS), lh * _R + rv, _DUMMY)
                ft_buf[j, dsl] = ft
                fh_buf[j, dsl] = fh
                return 0

            lax.fori_loop(0, _CHUNK // 16, vec_body, 0)

            w_buf = wb[p]
            descs = [
                pltpu.async_copy(w_buf.at[j], acc.at[ix.at[j]], scsem[p],
                                 add=True)
                for j in range(_SUBROWS)
                for ix in (ft_buf, fh_buf)
            ]
            for d in descs:
                d.wait()

        plsc.subcore_barrier()

        woff = pl.multiple_of(s * _WSLICE, 128)
        pltpu.sync_copy(acc.at[pl.ds(woff, _WSLICE)],
                        out.at[rid, pl.ds(woff, _WSLICE)])
        plsc.subcore_barrier()


def _relval_body(rf_ref, wt_ref, b_ref, o_ref):
    o_ref[...] = (
        jnp.dot(rf_ref[...], wt_ref[...], preferred_element_type=jnp.float32)
        + b_ref[...]
    )


def _mm_relu_body(c_ref, rv_ref, o_ref):
    o_ref[...] = jnp.maximum(
        jnp.dot(c_ref[...], rv_ref[...], preferred_element_type=jnp.float32),
        0.0,
    )


def kernel(local_entity, batch_heads, batch_rels, batch_tails, batch_ids,
           fact_ids, weight_list, weight_rel_list, rel_features, W, b):
    del local_entity, batch_ids, fact_ids, weight_list

    pad = _E_PAD - _E
    zi = jnp.zeros((pad,), jnp.int32)
    tails = jnp.concatenate([batch_tails.astype(jnp.int32), zi])
    heads = jnp.concatenate([batch_heads.astype(jnp.int32), zi])
    rels = jnp.concatenate([batch_rels.astype(jnp.int32), zi])
    w2d = jnp.concatenate(
        [weight_rel_list.astype(jnp.float32), jnp.zeros((pad,), jnp.float32)]
    ).reshape(_E_PAD // 128, 128)
    zeros_hbm = jnp.zeros((_ZSLICE,), jnp.float32)

    c_flat = _build_c(tails, heads, rels, w2d, zeros_hbm)
    C = c_flat.reshape(_N_ENT, _R)

    rel_val = pl.pallas_call(
        _relval_body,
        out_shape=jax.ShapeDtypeStruct((_R, _D), jnp.float32),
    )(rel_features.astype(jnp.float32), W.astype(jnp.float32).T,
      b.astype(jnp.float32).reshape(1, _D))

    rows_blk = 2000
    out = pl.pallas_call(
        _mm_relu_body,
        grid=(_N_ENT // rows_blk,),
        in_specs=[
            pl.BlockSpec((rows_blk, _R), lambda i: (i, 0)),
            pl.BlockSpec((_R, _D), lambda i: (0, 0)),
        ],
        out_specs=pl.BlockSpec((rows_blk, _D), lambda i: (i, 0)),
        out_shape=jax.ShapeDtypeStruct((_N_ENT, _D), jnp.float32),
    )(C, rel_val)

    return out.reshape(_B, _L, _D)


# speedup vs baseline: 6.1542x; 6.1542x over previous
import functools

import jax
import jax.numpy as jnp
from jax import lax
from jax.experimental import pallas as pl
from jax.experimental.pallas import tpu as pltpu
from jax.experimental.pallas import tpu_sc as plsc

_B, _L, _E, _R, _D = 10, 1000, 320000, 512, 128
_N_ENT = _B * _L

_N_CORES = 2
_N_TILES = 16
_N_RANGES = 4
_ROWS = _N_ENT // _N_RANGES
_ACC_WORDS = _ROWS * _R
_ACC_PAD = _ACC_WORDS + 2048
_ZSLICE = _ACC_PAD // _N_TILES
_WSLICE = _ACC_WORDS // _N_TILES
_DUMMY = _ACC_WORDS

_CHUNK = 4096
_SUBROWS = _CHUNK // 128
_Q = 20480
_NCHUNK = _Q // _CHUNK
_E_PAD = _N_TILES * _Q


_sc_mesh = plsc.VectorSubcoreMesh(core_axis_name="c", subcore_axis_name="s")


@functools.partial(
    pl.kernel,
    out_type=jax.ShapeDtypeStruct((_N_RANGES, _ACC_WORDS), jnp.float32),
    mesh=_sc_mesh,
    scratch_types=[
        [pltpu.VMEM((_CHUNK,), jnp.int32)] * 2,
        [pltpu.VMEM((_CHUNK,), jnp.int32)] * 2,
        [pltpu.VMEM((_CHUNK,), jnp.int32)] * 2,
        [pltpu.VMEM((_SUBROWS, 128), jnp.float32)] * 2,
        [pltpu.VMEM((_SUBROWS, 128), jnp.int32)] * 2,
        [pltpu.VMEM((_SUBROWS, 128), jnp.int32)] * 2,
        pltpu.VMEM_SHARED((_ACC_PAD,), jnp.float32),
        [pltpu.SemaphoreType.DMA] * 2,
        [pltpu.SemaphoreType.DMA] * 2,
    ],
)
def _build_c(tails, heads, rels, w2d, zeros_hbm, out,
             tb, hb, rb, wb, ftb, fhb, acc, insem, scsem):
    c = lax.axis_index("c")
    s = lax.axis_index("s")

    def fire_inputs(k, p):
        off = s * _Q + k * _CHUNK
        row_off = pl.multiple_of(off // 128, 8)
        return [
            pltpu.async_copy(tails.at[pl.ds(off, _CHUNK)], tb[p], insem[p]),
            pltpu.async_copy(heads.at[pl.ds(off, _CHUNK)], hb[p], insem[p]),
            pltpu.async_copy(rels.at[pl.ds(off, _CHUNK)], rb[p], insem[p]),
            pltpu.async_copy(w2d.at[pl.ds(row_off, _SUBROWS)], wb[p], insem[p]),
        ]

    for rng in range(_N_RANGES // _N_CORES):
        rid = c * (_N_RANGES // _N_CORES) + rng
        base_row = rid * _ROWS

        pltpu.sync_copy(zeros_hbm,
                        acc.at[pl.ds(pl.multiple_of(s * _ZSLICE, 128), _ZSLICE)])
        plsc.subcore_barrier()

        for k in range(_NCHUNK):
            p = 0
            for d in fire_inputs(k, p):
                d.wait()

            t_buf, h_buf, r_buf = tb[p], hb[p], rb[p]
            ft_buf, fh_buf = ftb[p], fhb[p]

            def vec_body(i, _):
                j = i // 8
                l = i - j * 8
                sl = pl.ds(i * 16, 16)
                dsl = pl.ds(l * 16, 16)
                tv = t_buf[sl]
                hv = h_buf[sl]
                rv = r_buf[sl]
                dmy = _DUMMY + rv
                lt = tv - base_row
                ft = jnp.where((lt >= 0) & (lt < _ROWS), lt * _R + rv, dmy)
                lh = hv - base_row
                fh = jnp.where((lh >= 0) & (lh < _ROWS), lh * _R + rv, dmy)
                ft_buf[j, dsl] = ft
                fh_buf[j, dsl] = fh
                return 0

            lax.fori_loop(0, _CHUNK // 16, vec_body, 0)

            w_buf = wb[p]
            descs = [
                pltpu.async_copy(w_buf.at[j], acc.at[ix.at[j]], scsem[p],
                                 add=True)
                for j in range(_SUBROWS)
                for ix in (ft_buf, fh_buf)
            ]
            for d in descs:
                d.wait()

        plsc.subcore_barrier()

        woff = pl.multiple_of(s * _WSLICE, 128)
        pltpu.sync_copy(acc.at[pl.ds(woff, _WSLICE)],
                        out.at[rid, pl.ds(woff, _WSLICE)])
        plsc.subcore_barrier()


def _relval_body(rf_ref, wt_ref, b_ref, o_ref):
    o_ref[...] = (
        jnp.dot(rf_ref[...], wt_ref[...], preferred_element_type=jnp.float32)
        + b_ref[...]
    )


def _mm_relu_body(c_ref, rv_ref, o_ref):
    o_ref[...] = jnp.maximum(
        jnp.dot(c_ref[...], rv_ref[...], preferred_element_type=jnp.float32),
        0.0,
    )


def kernel(local_entity, batch_heads, batch_rels, batch_tails, batch_ids,
           fact_ids, weight_list, weight_rel_list, rel_features, W, b):
    del local_entity, batch_ids, fact_ids, weight_list

    pad = _E_PAD - _E
    zi = jnp.zeros((pad,), jnp.int32)
    tails = jnp.concatenate([batch_tails.astype(jnp.int32), zi])
    heads = jnp.concatenate([batch_heads.astype(jnp.int32), zi])
    rels = jnp.concatenate([batch_rels.astype(jnp.int32), zi])
    w2d = jnp.concatenate(
        [weight_rel_list.astype(jnp.float32), jnp.zeros((pad,), jnp.float32)]
    ).reshape(_E_PAD // 128, 128)
    zeros_hbm = jnp.zeros((_ZSLICE,), jnp.float32)

    c_flat = _build_c(tails, heads, rels, w2d, zeros_hbm)
    C = c_flat.reshape(_N_ENT, _R)

    rel_val = pl.pallas_call(
        _relval_body,
        out_shape=jax.ShapeDtypeStruct((_R, _D), jnp.float32),
    )(rel_features.astype(jnp.float32), W.astype(jnp.float32).T,
      b.astype(jnp.float32).reshape(1, _D))

    rows_blk = 2000
    out = pl.pallas_call(
        _mm_relu_body,
        grid=(_N_ENT // rows_blk,),
        in_specs=[
            pl.BlockSpec((rows_blk, _R), lambda i: (i, 0)),
            pl.BlockSpec((_R, _D), lambda i: (0, 0)),
        ],
        out_specs=pl.BlockSpec((rows_blk, _D), lambda i: (i, 0)),
        out_shape=jax.ShapeDtypeStruct((_N_ENT, _D), jnp.float32),
    )(C, rel_val)

    return out.reshape(_B, _L, _D)


# speedup vs baseline: 7.6335x; 1.2404x over previous
import functools

import jax
import jax.numpy as jnp
from jax import lax
from jax.experimental import pallas as pl
from jax.experimental.pallas import tpu as pltpu
from jax.experimental.pallas import tpu_sc as plsc

_B, _L, _E, _R, _D = 10, 1000, 320000, 512, 128
_N_ENT = _B * _L

_N_CORES = 2
_N_TILES = 16
_N_RANGES = 4
_ROWS = _N_ENT // _N_RANGES
_ACC_WORDS = _ROWS * _R
_ACC_PAD = _ACC_WORDS + 2048
_ZSLICE = _ACC_PAD // _N_TILES
_WSLICE = _ACC_WORDS // _N_TILES
_DUMMY = _ACC_WORDS

_CHUNK = 4096
_SUBROWS = _CHUNK // 128
_Q = 20480
_NCHUNK = _Q // _CHUNK
_E_PAD = _N_TILES * _Q


_sc_mesh = plsc.VectorSubcoreMesh(core_axis_name="c", subcore_axis_name="s")


@functools.partial(
    pl.kernel,
    out_type=jax.ShapeDtypeStruct((8, _ACC_WORDS // 2), jnp.float32),
    mesh=_sc_mesh,
    scratch_types=[
        [pltpu.VMEM((_CHUNK,), jnp.int32)] * 2,
        [pltpu.VMEM((_CHUNK,), jnp.int32)] * 2,
        [pltpu.VMEM((_CHUNK,), jnp.int32)] * 2,
        [pltpu.VMEM((_SUBROWS, 128), jnp.float32)] * 2,
        [pltpu.VMEM((_SUBROWS, 128), jnp.int32)] * 2,
        [pltpu.VMEM((_SUBROWS, 128), jnp.int32)] * 2,
        pltpu.VMEM_SHARED((_ACC_PAD,), jnp.float32),
        [pltpu.SemaphoreType.DMA] * 2,
        [pltpu.SemaphoreType.DMA] * 2,
    ],
)
def _build_c(tails, heads, rels, w2d, zeros_hbm, out,
             tb, hb, rb, wb, ftb, fhb, acc, insem, scsem):
    c = lax.axis_index("c")
    s = lax.axis_index("s")

    def fire_inputs(k, p):
        off = s * _Q + k * _CHUNK
        row_off = pl.multiple_of(off // 128, 8)
        return [
            pltpu.async_copy(tails.at[pl.ds(off, _CHUNK)], tb[p], insem[p]),
            pltpu.async_copy(heads.at[pl.ds(off, _CHUNK)], hb[p], insem[p]),
            pltpu.async_copy(rels.at[pl.ds(off, _CHUNK)], rb[p], insem[p]),
            pltpu.async_copy(w2d.at[pl.ds(row_off, _SUBROWS)], wb[p], insem[p]),
        ]

    for rng in range(_N_RANGES // _N_CORES):
        rid = c * (_N_RANGES // _N_CORES) + rng
        base_row = rid * _ROWS

        pltpu.sync_copy(zeros_hbm,
                        acc.at[pl.ds(pl.multiple_of(s * _ZSLICE, 128), _ZSLICE)])
        plsc.subcore_barrier()

        for k in range(_NCHUNK):
            p = 0
            for d in fire_inputs(k, p):
                d.wait()

            t_buf, h_buf, r_buf = tb[p], hb[p], rb[p]
            ft_buf, fh_buf = ftb[p], fhb[p]

            def vec_body(i, _):
                j = i // 8
                l = i - j * 8
                sl = pl.ds(i * 16, 16)
                dsl = pl.ds(l * 16, 16)
                tv = t_buf[sl]
                hv = h_buf[sl]
                rv = r_buf[sl]
                dmy = _DUMMY + rv
                lt = tv - base_row
                ft = jnp.where((lt >= 0) & (lt < _ROWS), lt * _R + rv, dmy)
                lh = hv - base_row
                fh = jnp.where((lh >= 0) & (lh < _ROWS), lh * _R + rv, dmy)
                ft_buf[j, dsl] = ft
                fh_buf[j, dsl] = fh
                return 0

            lax.fori_loop(0, _CHUNK // 16, vec_body, 0)

            w_buf = wb[p]
            descs = [
                pltpu.async_copy(w_buf.at[j], acc.at[ix.at[j]], scsem[p],
                                 add=True)
                for j in range(_SUBROWS)
                for ix in (ft_buf, fh_buf)
            ]
            for d in descs:
                d.wait()

        plsc.subcore_barrier()

        woff = pl.multiple_of(s * _WSLICE, 128)
        orow = 2 * rid + s // 8
        ocol = pl.multiple_of((s - (s // 8) * 8) * _WSLICE, 128)
        pltpu.sync_copy(acc.at[pl.ds(woff, _WSLICE)],
                        out.at[orow, pl.ds(ocol, _WSLICE)])
        plsc.subcore_barrier()


def _relval_body(rf_ref, wt_ref, b_ref, o_ref):
    o_ref[...] = (
        jnp.dot(rf_ref[...], wt_ref[...], preferred_element_type=jnp.float32)
        + b_ref[...]
    )


def _mm_relu_body(c_ref, rv_ref, o_ref):
    o_ref[...] = jnp.maximum(
        jnp.dot(c_ref[...], rv_ref[...], preferred_element_type=jnp.float32),
        0.0,
    )


def kernel(local_entity, batch_heads, batch_rels, batch_tails, batch_ids,
           fact_ids, weight_list, weight_rel_list, rel_features, W, b):
    del local_entity, batch_ids, fact_ids, weight_list

    pad = _E_PAD - _E
    zi = jnp.zeros((pad,), jnp.int32)
    tails = jnp.concatenate([batch_tails.astype(jnp.int32), zi])
    heads = jnp.concatenate([batch_heads.astype(jnp.int32), zi])
    rels = jnp.concatenate([batch_rels.astype(jnp.int32), zi])
    w2d = jnp.concatenate(
        [weight_rel_list.astype(jnp.float32), jnp.zeros((pad,), jnp.float32)]
    ).reshape(_E_PAD // 128, 128)
    zeros_hbm = jnp.zeros((_ZSLICE,), jnp.float32)

    c_flat = _build_c(tails, heads, rels, w2d, zeros_hbm)
    C = c_flat.reshape(_N_ENT, _R)

    rel_val = pl.pallas_call(
        _relval_body,
        out_shape=jax.ShapeDtypeStruct((_R, _D), jnp.float32),
    )(rel_features.astype(jnp.float32), W.astype(jnp.float32).T,
      b.astype(jnp.float32).reshape(1, _D))

    rows_blk = 2000
    out = pl.pallas_call(
        _mm_relu_body,
        grid=(_N_ENT // rows_blk,),
        in_specs=[
            pl.BlockSpec((rows_blk, _R), lambda i: (i, 0)),
            pl.BlockSpec((_R, _D), lambda i: (0, 0)),
        ],
        out_specs=pl.BlockSpec((rows_blk, _D), lambda i: (i, 0)),
        out_shape=jax.ShapeDtypeStruct((_N_ENT, _D), jnp.float32),
    )(C, rel_val)

    return out.reshape(_B, _L, _D)


# speedup vs baseline: 8.1236x; 1.0642x over previous
import functools

import jax
import jax.numpy as jnp
from jax import lax
from jax.experimental import pallas as pl
from jax.experimental.pallas import tpu as pltpu
from jax.experimental.pallas import tpu_sc as plsc

_B, _L, _E, _R, _D = 10, 1000, 320000, 512, 128
_N_ENT = _B * _L

_N_CORES = 2
_N_TILES = 16
_N_RANGES = 4
_ROWS = _N_ENT // _N_RANGES
_ACC_WORDS = _ROWS * _R
_ACC_PAD = _ACC_WORDS + 2048
_ZSLICE = _ACC_PAD // _N_TILES
_WSLICE = _ACC_WORDS // _N_TILES
_DUMMY = _ACC_WORDS

_CHUNK = 4096
_SUBROWS = _CHUNK // 128
_Q = 20480
_NCHUNK = _Q // _CHUNK
_E_PAD = _N_TILES * _Q


_sc_mesh = plsc.VectorSubcoreMesh(core_axis_name="c", subcore_axis_name="s")


@functools.partial(
    pl.kernel,
    out_type=jax.ShapeDtypeStruct((8, _ACC_WORDS // 2), jnp.float32),
    mesh=_sc_mesh,
    scratch_types=[
        [pltpu.VMEM((_CHUNK,), jnp.int32)] * 2,
        [pltpu.VMEM((_CHUNK,), jnp.int32)] * 2,
        [pltpu.VMEM((_CHUNK,), jnp.int32)] * 2,
        [pltpu.VMEM((_SUBROWS, 128), jnp.float32)] * 2,
        [pltpu.VMEM((_SUBROWS, 128), jnp.int32)] * 2,
        [pltpu.VMEM((_SUBROWS, 128), jnp.int32)] * 2,
        pltpu.VMEM_SHARED((_ACC_PAD,), jnp.float32),
        [pltpu.SemaphoreType.DMA] * 2,
        [pltpu.SemaphoreType.DMA] * 2,
    ],
)
def _build_c(tails, heads, rels, w2d, zeros_hbm, out,
             tb, hb, rb, wb, ftb, fhb, acc, insem, scsem):
    c = lax.axis_index("c")
    s = lax.axis_index("s")

    def fire_inputs(k, p):
        off = s * _Q + k * _CHUNK
        row_off = pl.multiple_of(off // 128, 8)
        return [
            pltpu.async_copy(tails.at[pl.ds(off, _CHUNK)], tb[p], insem[p]),
            pltpu.async_copy(heads.at[pl.ds(off, _CHUNK)], hb[p], insem[p]),
            pltpu.async_copy(rels.at[pl.ds(off, _CHUNK)], rb[p], insem[p]),
            pltpu.async_copy(w2d.at[pl.ds(row_off, _SUBROWS)], wb[p], insem[p]),
        ]

    for rng in range(_N_RANGES // _N_CORES):
        rid = c * (_N_RANGES // _N_CORES) + rng
        base_row = rid * _ROWS

        pltpu.sync_copy(zeros_hbm,
                        acc.at[pl.ds(pl.multiple_of(s * _ZSLICE, 128), _ZSLICE)])
        plsc.subcore_barrier()

        in_descs = {0: fire_inputs(0, 0)}
        for k in range(_NCHUNK):
            p = k % 2
            for d in in_descs.pop(k):
                d.wait()
            if k + 1 < _NCHUNK:
                in_descs[k + 1] = fire_inputs(k + 1, 1 - p)

            t_buf, h_buf, r_buf = tb[p], hb[p], rb[p]
            ft_buf, fh_buf = ftb[p], fhb[p]

            def vec_body(i, _):
                j = i // 8
                l = i - j * 8
                sl = pl.ds(i * 16, 16)
                dsl = pl.ds(l * 16, 16)
                tv = t_buf[sl]
                hv = h_buf[sl]
                rv = r_buf[sl]
                lt = tv - base_row
                ft = jnp.where((lt >= 0) & (lt < _ROWS), lt * _R + rv,
                               _DUMMY + (rv * 4 + (tv & 3)))
                lh = hv - base_row
                fh = jnp.where((lh >= 0) & (lh < _ROWS), lh * _R + rv,
                               _DUMMY + (rv * 4 + (hv & 3)))
                ft_buf[j, dsl] = ft
                fh_buf[j, dsl] = fh
                return 0

            lax.fori_loop(0, _CHUNK // 16, vec_body, 0)

            w_buf = wb[p]
            descs = [
                pltpu.async_copy(w_buf.at[j], acc.at[ix.at[j]], scsem[p],
                                 add=True)
                for j in range(_SUBROWS)
                for ix in (ft_buf, fh_buf)
            ]
            for d in descs:
                d.wait()

        plsc.subcore_barrier()

        woff = pl.multiple_of(s * _WSLICE, 128)
        orow = 2 * rid + s // 8
        ocol = pl.multiple_of((s - (s // 8) * 8) * _WSLICE, 128)
        pltpu.sync_copy(acc.at[pl.ds(woff, _WSLICE)],
                        out.at[orow, pl.ds(ocol, _WSLICE)])
        plsc.subcore_barrier()


def _relval_body(rf_ref, wt_ref, b_ref, o_ref):
    o_ref[...] = (
        jnp.dot(rf_ref[...], wt_ref[...], preferred_element_type=jnp.float32)
        + b_ref[...]
    )


def _mm_relu_body(c_ref, rv_ref, o_ref):
    o_ref[...] = jnp.maximum(
        jnp.dot(c_ref[...], rv_ref[...], preferred_element_type=jnp.float32),
        0.0,
    )


def kernel(local_entity, batch_heads, batch_rels, batch_tails, batch_ids,
           fact_ids, weight_list, weight_rel_list, rel_features, W, b):
    del local_entity, batch_ids, fact_ids, weight_list

    pad = _E_PAD - _E
    zi = jnp.zeros((pad,), jnp.int32)
    tails = jnp.concatenate([batch_tails.astype(jnp.int32), zi])
    heads = jnp.concatenate([batch_heads.astype(jnp.int32), zi])
    rels = jnp.concatenate([batch_rels.astype(jnp.int32), zi])
    w2d = jnp.concatenate(
        [weight_rel_list.astype(jnp.float32), jnp.zeros((pad,), jnp.float32)]
    ).reshape(_E_PAD // 128, 128)
    zeros_hbm = jnp.zeros((_ZSLICE,), jnp.float32)

    c_flat = _build_c(tails, heads, rels, w2d, zeros_hbm)
    C = c_flat.reshape(_N_ENT, _R)

    rel_val = pl.pallas_call(
        _relval_body,
        out_shape=jax.ShapeDtypeStruct((_R, _D), jnp.float32),
    )(rel_features.astype(jnp.float32), W.astype(jnp.float32).T,
      b.astype(jnp.float32).reshape(1, _D))

    rows_blk = 2000
    out = pl.pallas_call(
        _mm_relu_body,
        grid=(_N_ENT // rows_blk,),
        in_specs=[
            pl.BlockSpec((rows_blk, _R), lambda i: (i, 0)),
            pl.BlockSpec((_R, _D), lambda i: (0, 0)),
        ],
        out_specs=pl.BlockSpec((rows_blk, _D), lambda i: (i, 0)),
        out_shape=jax.ShapeDtypeStruct((_N_ENT, _D), jnp.float32),
    )(C, rel_val)

    return out.reshape(_B, _L, _D)


# speedup vs baseline: 8.8603x; 1.0907x over previous
import functools

import jax
import jax.numpy as jnp
from jax import lax
from jax.experimental import pallas as pl
from jax.experimental.pallas import tpu as pltpu
from jax.experimental.pallas import tpu_sc as plsc

_B, _L, _E, _R, _D = 10, 1000, 320000, 512, 128
_N_ENT = _B * _L

_N_CORES = 2
_N_TILES = 16
_N_RANGES = 4
_ROWS = _N_ENT // _N_RANGES
_ACC_WORDS = _ROWS * _R
_ACC_PAD = _ACC_WORDS + 2048
_ZSLICE = _ACC_PAD // _N_TILES
_WSLICE = _ACC_WORDS // _N_TILES
_DUMMY = _ACC_WORDS

_CHUNK = 4096
_SUBROWS = _CHUNK // 128
_Q = 20480
_NCHUNK = _Q // _CHUNK
_E_PAD = _N_TILES * _Q


_sc_mesh = plsc.VectorSubcoreMesh(core_axis_name="c", subcore_axis_name="s")


@functools.partial(
    pl.kernel,
    out_type=jax.ShapeDtypeStruct((8, _ACC_WORDS // 2), jnp.float32),
    mesh=_sc_mesh,
    scratch_types=[
        [pltpu.VMEM((_CHUNK,), jnp.int32)] * 2,
        [pltpu.VMEM((_CHUNK,), jnp.int32)] * 2,
        [pltpu.VMEM((_CHUNK,), jnp.int32)] * 2,
        [pltpu.VMEM((_SUBROWS, 128), jnp.float32)] * 2,
        [pltpu.VMEM((_SUBROWS, 128), jnp.int32)] * 2,
        [pltpu.VMEM((_SUBROWS, 128), jnp.int32)] * 2,
        pltpu.VMEM_SHARED((_ACC_PAD,), jnp.float32),
        [pltpu.SemaphoreType.DMA] * 2,
        [pltpu.SemaphoreType.DMA] * 2,
    ],
)
def _build_c(tails, heads, rels, w2d, zeros_hbm, out,
             tb, hb, rb, wb, ftb, fhb, acc, insem, scsem):
    c = lax.axis_index("c")
    s = lax.axis_index("s")

    def fire_inputs(k, p):
        off = s * _Q + k * _CHUNK
        row_off = pl.multiple_of(off // 128, 8)
        return [
            pltpu.async_copy(tails.at[pl.ds(off, _CHUNK)], tb[p], insem[p]),
            pltpu.async_copy(heads.at[pl.ds(off, _CHUNK)], hb[p], insem[p]),
            pltpu.async_copy(rels.at[pl.ds(off, _CHUNK)], rb[p], insem[p]),
            pltpu.async_copy(w2d.at[pl.ds(row_off, _SUBROWS)], wb[p], insem[p]),
        ]

    for rng in range(_N_RANGES // _N_CORES):
        rid = c * (_N_RANGES // _N_CORES) + rng
        base_row = rid * _ROWS

        pltpu.sync_copy(zeros_hbm,
                        acc.at[pl.ds(pl.multiple_of(s * _ZSLICE, 128), _ZSLICE)])
        plsc.subcore_barrier()

        in_descs = {0: fire_inputs(0, 0)}
        for k in range(_NCHUNK):
            p = k % 2
            for d in in_descs.pop(k):
                d.wait()
            if k + 1 < _NCHUNK:
                in_descs[k + 1] = fire_inputs(k + 1, 1 - p)

            t_buf, h_buf, r_buf = tb[p], hb[p], rb[p]
            ft_buf, fh_buf = ftb[p], fhb[p]

            def vec_body(i, _):
                j = i // 8
                l = i - j * 8
                sl = pl.ds(i * 16, 16)
                dsl = pl.ds(l * 16, 16)
                tv = t_buf[sl]
                hv = h_buf[sl]
                rv = r_buf[sl]
                lt = tv - base_row
                ft = jnp.where((lt >= 0) & (lt < _ROWS), lt * _R + rv,
                               _DUMMY + (rv * 4 + (tv & 3)))
                lh = hv - base_row
                fh = jnp.where((lh >= 0) & (lh < _ROWS), lh * _R + rv,
                               _DUMMY + (rv * 4 + (hv & 3)))
                ft_buf[j, dsl] = ft
                fh_buf[j, dsl] = fh
                return 0

            lax.fori_loop(0, _CHUNK // 16, vec_body, 0)

            w_buf = wb[p]
            descs = [
                pltpu.async_copy(w_buf.at[j], acc.at[ix.at[j]], scsem[p],
                                 add=True)
                for j in range(_SUBROWS)
                for ix in (ft_buf, fh_buf)
            ]
            for d in descs:
                d.wait()

        plsc.subcore_barrier()

        woff = pl.multiple_of(s * _WSLICE, 128)
        orow = 2 * rid + s // 8
        ocol = pl.multiple_of((s - (s // 8) * 8) * _WSLICE, 128)
        pltpu.sync_copy(acc.at[pl.ds(woff, _WSLICE)],
                        out.at[orow, pl.ds(ocol, _WSLICE)])
        plsc.subcore_barrier()


def _relval_body(rf_ref, wt_ref, b_ref, o_ref):
    o_ref[...] = (
        jnp.dot(rf_ref[...], wt_ref[...], preferred_element_type=jnp.float32)
        + b_ref[...]
    )


def _mm_relu_body(c_ref, rv_ref, o_ref):
    cmat = c_ref[...].reshape(_N_ENT, _R)
    o_ref[...] = jnp.maximum(
        jnp.dot(cmat, rv_ref[...], preferred_element_type=jnp.float32),
        0.0,
    )


def kernel(local_entity, batch_heads, batch_rels, batch_tails, batch_ids,
           fact_ids, weight_list, weight_rel_list, rel_features, W, b):
    del local_entity, batch_ids, fact_ids, weight_list

    pad = _E_PAD - _E
    zi = jnp.zeros((pad,), jnp.int32)
    tails = jnp.concatenate([batch_tails.astype(jnp.int32), zi])
    heads = jnp.concatenate([batch_heads.astype(jnp.int32), zi])
    rels = jnp.concatenate([batch_rels.astype(jnp.int32), zi])
    w2d = jnp.concatenate(
        [weight_rel_list.astype(jnp.float32), jnp.zeros((pad,), jnp.float32)]
    ).reshape(_E_PAD // 128, 128)
    zeros_hbm = jnp.zeros((_ZSLICE,), jnp.float32)

    c_flat = _build_c(tails, heads, rels, w2d, zeros_hbm)

    rel_val = pl.pallas_call(
        _relval_body,
        out_shape=jax.ShapeDtypeStruct((_R, _D), jnp.float32),
    )(rel_features.astype(jnp.float32), W.astype(jnp.float32).T,
      b.astype(jnp.float32).reshape(1, _D))

    out = pl.pallas_call(
        _mm_relu_body,
        out_shape=jax.ShapeDtypeStruct((_N_ENT, _D), jnp.float32),
    )(c_flat, rel_val)

    return out.reshape(_B, _L, _D)
